# baseline (speedup 1.0000x reference)
import functools

import jax
import jax.numpy as jnp
from jax import lax
from jax.experimental import pallas as pl
from jax.experimental.pallas import tpu as pltpu
from jax.experimental.pallas import tpu_sc as plsc

_NC = 2
_NS = 16
_NW = _NC * _NS
_G = 128


@jax.jit
def kernel(indices, table):
    (B,) = indices.shape
    V, D = table.shape
    b_per_w = B // _NW
    n_g = b_per_w // _G

    idx3 = indices.reshape(_NW, n_g, _G).astype(jnp.int32)

    mesh = plsc.VectorSubcoreMesh(core_axis_name="c", subcore_axis_name="s")

    @functools.partial(
        pl.kernel,
        mesh=mesh,
        out_type=jax.ShapeDtypeStruct((B, D), jnp.float32),
        scratch_types=[
            pltpu.VMEM((n_g, _G), jnp.int32),
            pltpu.VMEM((_G, D), jnp.float32),
            pltpu.SemaphoreType.DMA,
        ],
        compiler_params=pltpu.CompilerParams(use_tc_tiling_on_sc=False),
    )
    def emb(idx_hbm, table_hbm, out_hbm, idx_v, rows_v, sem):
        wid = lax.axis_index("s") * _NC + lax.axis_index("c")
        base = wid * b_per_w
        pltpu.sync_copy(idx_hbm.at[wid], idx_v)

        def body(g, _):
            pltpu.async_copy(table_hbm.at[idx_v.at[g]], rows_v, sem).wait()
            pltpu.sync_copy(rows_v, out_hbm.at[pl.ds(base + g * _G, _G)])
            return 0

        lax.fori_loop(0, n_g, body, 0)

    return emb(idx3, table)


# speedup vs baseline: 1.1547x; 1.1547x over previous
import functools

import jax
import jax.numpy as jnp
from jax import lax
from jax.experimental import pallas as pl
from jax.experimental.pallas import tpu as pltpu
from jax.experimental.pallas import tpu_sc as plsc

_NC = 2
_NS = 16
_NW = _NC * _NS
_G = 128


@jax.jit
def kernel(indices, table):
    (B,) = indices.shape
    V, D = table.shape
    b_per_w = B // _NW
    n_g = b_per_w // _G

    k_per_chunk = 20
    C = k_per_chunk * _G
    n_chunks = b_per_w // C

    idx3 = indices.reshape(_NW, n_g, _G).astype(jnp.int32)

    mesh = plsc.VectorSubcoreMesh(core_axis_name="c", subcore_axis_name="s")

    @functools.partial(
        pl.kernel,
        mesh=mesh,
        out_type=jax.ShapeDtypeStruct((B, D), jnp.float32),
        scratch_types=[
            pltpu.VMEM((n_g, _G), jnp.int32),
            pltpu.VMEM((C, D), jnp.float32),
            pltpu.SemaphoreType.DMA,
        ],
        compiler_params=pltpu.CompilerParams(use_tc_tiling_on_sc=False),
    )
    def emb(idx_hbm, table_hbm, out_hbm, idx_v, rows_v, sem):
        wid = lax.axis_index("s") * _NC + lax.axis_index("c")
        base = wid * b_per_w
        pltpu.sync_copy(idx_hbm.at[wid], idx_v)

        def body(c, _):
            copies = []
            for j in range(k_per_chunk):
                copies.append(pltpu.async_copy(
                    table_hbm.at[idx_v.at[c * k_per_chunk + j]],
                    rows_v.at[pl.ds(j * _G, _G)],
                    sem,
                ))
            for cp in copies:
                cp.wait()
            pltpu.sync_copy(rows_v, out_hbm.at[pl.ds(base + c * C, C)])
            return 0

        lax.fori_loop(0, n_chunks, body, 0)

    return emb(idx3, table)


# speedup vs baseline: 1.1647x; 1.0086x over previous
import functools

import jax
import jax.numpy as jnp
from jax import lax
from jax.experimental import pallas as pl
from jax.experimental.pallas import tpu as pltpu
from jax.experimental.pallas import tpu_sc as plsc

_NC = 2
_NS = 16
_NW = _NC * _NS
_NBUF = 3
_C = 1600


@jax.jit
def kernel(indices, table):
    (B,) = indices.shape
    V, D = table.shape
    b_per_w = B // _NW
    n_chunks = b_per_w // _C

    idx2 = indices.reshape(_NW, b_per_w).astype(jnp.int32)

    mesh = plsc.VectorSubcoreMesh(core_axis_name="c", subcore_axis_name="s")

    @functools.partial(
        pl.kernel,
        mesh=mesh,
        out_type=jax.ShapeDtypeStruct((B, D), jnp.float32),
        scratch_types=[
            pltpu.VMEM((b_per_w,), jnp.int32),
            pltpu.VMEM((_NBUF, _C, D), jnp.float32),
            [pltpu.SemaphoreType.DMA] * _NBUF,
            [pltpu.SemaphoreType.DMA] * _NBUF,
        ],
        compiler_params=pltpu.CompilerParams(use_tc_tiling_on_sc=False),
    )
    def emb(idx_hbm, table_hbm, out_hbm, idx_v, rows_v, gsems, ssems):
        wid = lax.axis_index("s") * _NC + lax.axis_index("c")
        base = wid * b_per_w
        pltpu.sync_copy(idx_hbm.at[wid], idx_v)

        def fire_gather(g, b):
            return pltpu.async_copy(
                table_hbm.at[idx_v.at[pl.ds(g * _C, _C)]],
                rows_v.at[b], gsems[b])

        def fire_store(g, b):
            return pltpu.async_copy(
                rows_v.at[b], out_hbm.at[pl.ds(base + g * _C, _C)], ssems[b])

        gath = [None] * _NBUF
        stor = [None] * _NBUF
        for g in range(n_chunks):
            b = g % _NBUF
            if stor[b] is not None:
                stor[b].wait()
            gath[b] = fire_gather(g, b)
            c = g - (_NBUF - 1)
            if c >= 0:
                bc = c % _NBUF
                gath[bc].wait()
                stor[bc] = fire_store(c, bc)
        for c in range(max(0, n_chunks - (_NBUF - 1)), n_chunks):
            bc = c % _NBUF
            gath[bc].wait()
            stor[bc] = fire_store(c, bc)
        for b in range(_NBUF):
            if stor[b] is not None:
                stor[b].wait()

    return emb(idx2, table)


# speedup vs baseline: 1.5052x; 1.2923x over previous
import functools

import jax
import jax.numpy as jnp
from jax import lax
from jax.experimental import pallas as pl
from jax.experimental.pallas import tpu as pltpu
from jax.experimental.pallas import tpu_sc as plsc

_NC = 2
_NS = 16
_NW = _NC * _NS
_C = 1280
_NBUF = 2


@jax.jit
def kernel(indices, table):
    (B,) = indices.shape
    V, D = table.shape
    b_per_w = B // _NW
    n_chunks = b_per_w // _C
    n_t = _C // 128
    n_tc = B // 128

    idx1 = indices.astype(jnp.int32)

    mesh = plsc.VectorSubcoreMesh(core_axis_name="c", subcore_axis_name="s")

    @functools.partial(
        pl.kernel,
        mesh=mesh,
        out_type=jax.ShapeDtypeStruct((2, n_tc, 8, 128), jnp.float32),
        scratch_types=[
            pltpu.VMEM((b_per_w,), jnp.int32),
            pltpu.VMEM((_NBUF, _C, D), jnp.float32),
            pltpu.VMEM((_NBUF, 2, n_t, 8, 128), jnp.float32),
            [pltpu.SemaphoreType.DMA] * _NBUF,
            [pltpu.SemaphoreType.DMA] * _NBUF,
        ],
        compiler_params=pltpu.CompilerParams(
            use_tc_tiling_on_sc=False, needs_layout_passes=False),
    )
    def emb(idx_hbm, table_hbm, out_hbm, idx_v, rows_v, tile_v, gsems, ssems):
        wid = lax.axis_index("s") * _NC + lax.axis_index("c")
        base = wid * b_per_w
        pltpu.sync_copy(idx_hbm.at[pl.ds(base, b_per_w)], idx_v)

        iota = lax.iota(jnp.int32, 16)

        def fire_gather(g, b):
            return pltpu.async_copy(
                table_hbm.at[idx_v.at[pl.ds(g * _C, _C)]],
                rows_v.at[b], gsems[b])

        def fire_stores(g, b):
            tc0 = (base + g * _C) // 128
            return [
                pltpu.async_copy(tile_v.at[b, r],
                                 out_hbm.at[r, pl.ds(tc0, n_t)], ssems[b])
                for r in (0, 1)
            ]

        def retile(b):
            rows = rows_v.at[b]

            def body(t, _):
                k = t // 8
                s = lax.rem(t, 8)
                for r in (0, 1):
                    f = jnp.full((16,), 8 * r + s, jnp.int32)
                    for m in range(8):
                        ridx = 128 * k + 16 * m + iota
                        v = plsc.load_gather(rows, [ridx, f])
                        tile_v[b, r, k, s, pl.ds(16 * m, 16)] = v
                return 0

            lax.fori_loop(0, n_t * 8, body, 0)

        gath = [None] * _NBUF
        stor = [None] * _NBUF
        gath[0] = fire_gather(0, 0)
        for g in range(n_chunks):
            b = g % _NBUF
            nb = (g + 1) % _NBUF
            if g + 1 < n_chunks:
                if stor[nb] is not None:
                    for cp in stor[nb]:
                        cp.wait()
                    stor[nb] = None
                gath[nb] = fire_gather(g + 1, nb)
            gath[b].wait()
            retile(b)
            stor[b] = fire_stores(g, b)
        for b in range(_NBUF):
            if stor[b] is not None:
                for cp in stor[b]:
                    cp.wait()

    out4d = emb(idx1, table)
    return jnp.transpose(out4d, (1, 3, 0, 2)).reshape(B, D)
